# baseline (speedup 1.0000x reference)
import functools

import jax
import jax.numpy as jnp
from jax import lax
from jax.experimental import pallas as pl
from jax.experimental.pallas import tpu as pltpu
from jax.experimental.pallas import tpu_sc as plsc

B, S, N, E = 8, 256, 10000, 160000
C, H, O, M, NC = 64, 256, 128, 256, 2
NP = 10240
NW = 32
EPW = E // NW
EBLK = 100
NBLK = EPW // EBLK
ROWS_PW = NP // 16
NT = 512
RT = 256

_mesh = plsc.VectorSubcoreMesh(core_axis_name="c", subcore_axis_name="s")


def _zero_rows(zbuf_v, ncols):
    nregs = ncols // 16
    z = jnp.zeros((16,), jnp.float32)

    def body(i, _):
        for k in range(nregs):
            zbuf_v[i, pl.ds(k * 16, 16)] = z
        return 0

    lax.fori_loop(0, zbuf_v.shape[0], body, 0)


@functools.partial(
    pl.kernel, mesh=_mesh,
    out_type=jax.ShapeDtypeStruct((2, NP), jnp.float32),
    scratch_types=[
        pltpu.VMEM((NBLK, EBLK), jnp.int32),
        pltpu.VMEM((112,), jnp.float32),
        pltpu.VMEM((ROWS_PW,), jnp.float32),
        pltpu.VMEM_SHARED((NP,), jnp.float32),
    ],
)
def _deg_kernel(ei_hbm, out_hbm, idx_v, ones_v, zrow_v, acc_sh):
    c = lax.axis_index("c")
    s = lax.axis_index("s")
    w = c * 16 + s
    one = jnp.full((16,), 1.0, jnp.float32)
    z = jnp.zeros((16,), jnp.float32)
    for i in range(7):
        ones_v[pl.ds(i * 16, 16)] = one
    for i in range(ROWS_PW // 16):
        zrow_v[pl.ds(i * 16, 16)] = z
    pltpu.sync_copy(zrow_v, acc_sh.at[pl.ds(s * ROWS_PW, ROWS_PW)])
    plsc.subcore_barrier()
    pltpu.sync_copy(ei_hbm.at[1, w], idx_v)

    def blk(j, _):
        pltpu.sync_copy(ones_v.at[pl.ds(0, EBLK)], acc_sh.at[idx_v.at[j]],
                        add=True)
        return 0

    lax.fori_loop(0, NBLK, blk, 0)
    plsc.subcore_barrier()
    pltpu.sync_copy(acc_sh.at[pl.ds(s * ROWS_PW, ROWS_PW)],
                    out_hbm.at[c, pl.ds(s * ROWS_PW, ROWS_PW)])


def _make_agg(f_chunks):
    @functools.partial(
        pl.kernel, mesh=_mesh,
        out_type=jax.ShapeDtypeStruct((2, f_chunks, NP, 128), jnp.float32),
        scratch_types=[
            pltpu.VMEM((NBLK, EBLK), jnp.int32),
            pltpu.VMEM((NBLK, EBLK), jnp.int32),
            pltpu.VMEM((EBLK, 128), jnp.float32),
            pltpu.VMEM((128, 128), jnp.float32),
            pltpu.VMEM_SHARED((NP, 128), jnp.float32),
        ],
    )
    def agg(v_hbm, ei_hbm, out_hbm, src_v, dst_v, buf_v, zbuf_v, acc_sh):
        c = lax.axis_index("c")
        s = lax.axis_index("s")
        w = c * 16 + s
        _zero_rows(zbuf_v, 128)
        pltpu.sync_copy(ei_hbm.at[0, w], src_v)
        pltpu.sync_copy(ei_hbm.at[1, w], dst_v)
        for ch in range(f_chunks):
            vch = v_hbm.at[ch]
            for zz in range(ROWS_PW // 128):
                pltpu.sync_copy(
                    zbuf_v, acc_sh.at[pl.ds(s * ROWS_PW + zz * 128, 128)])
            plsc.subcore_barrier()

            def blk(j, _):
                pltpu.sync_copy(vch.at[src_v.at[j]], buf_v)
                pltpu.sync_copy(buf_v, acc_sh.at[dst_v.at[j]], add=True)
                return 0

            lax.fori_loop(0, NBLK, blk, 0)
            plsc.subcore_barrier()
            pltpu.sync_copy(acc_sh.at[pl.ds(s * ROWS_PW, ROWS_PW)],
                            out_hbm.at[c, ch, pl.ds(s * ROWS_PW, ROWS_PW)])
    return agg


_agg4 = _make_agg(4)
_agg8 = _make_agg(8)


def _cnn_body(x_ref, w_ref, b_ref, o_ref):
    x2 = x_ref[0]
    zrow = jnp.zeros((1, NT), jnp.float32)
    xm1 = jnp.concatenate([zrow, x2[:-1, :]], axis=0)
    xp1 = jnp.concatenate([x2[1:, :], zrow], axis=0)
    inv_s = 1.0 / S
    for c in range(C):
        w0 = w_ref[c, 0, 0]
        w1 = w_ref[c, 0, 1]
        w2 = w_ref[c, 0, 2]
        h = xm1 * w0 + x2 * w1 + xp1 * w2 + b_ref[c]
        h = jnp.maximum(h, 0.0)
        o_ref[0, c, :] = jnp.sum(h, axis=0) * inv_s


def _run_cnn(x, conv_w, conv_b):
    grid = (B, NP // NT)
    return pl.pallas_call(
        _cnn_body,
        grid=grid,
        in_specs=[
            pl.BlockSpec((1, S, NT), lambda b, n: (b, 0, n)),
            pl.BlockSpec(memory_space=pltpu.SMEM),
            pl.BlockSpec(memory_space=pltpu.SMEM),
        ],
        out_specs=pl.BlockSpec((1, C, NT), lambda b, n: (b, 0, n)),
        out_shape=jax.ShapeDtypeStruct((B, C, NP), jnp.float32),
    )(x, conv_w, conv_b)


def _scale_body(degT_ref, featT_ref, v_ref, dis_ref):
    deg = degT_ref[:, 0:1] + degT_ref[:, 1:2] + 1.0
    dis = lax.rsqrt(deg)
    dis_ref[...] = dis
    vt = featT_ref[...] * dis
    for ch in range(4):
        v_ref[ch] = vt[:, ch * 128:(ch + 1) * 128]


def _run_scale(degT, featT):
    grid = (NP // RT,)
    return pl.pallas_call(
        _scale_body,
        grid=grid,
        in_specs=[
            pl.BlockSpec((RT, 2), lambda r: (r, 0)),
            pl.BlockSpec((RT, B * C), lambda r: (r, 0)),
        ],
        out_specs=[
            pl.BlockSpec((4, RT, 128), lambda r: (0, r, 0)),
            pl.BlockSpec((RT, 1), lambda r: (r, 0)),
        ],
        out_shape=[
            jax.ShapeDtypeStruct((4, NP, 128), jnp.float32),
            jax.ShapeDtypeStruct((NP, 1), jnp.float32),
        ],
    )(degT, featT)


def _mid_body(p_ref, v_ref, dis_ref, w1_ref, b1_ref, w2_ref, o_ref):
    dis = dis_ref[...]
    w1 = w1_ref[...]
    b1 = b1_ref[...]
    w2 = w2_ref[...]
    for ch in range(4):
        a = (p_ref[0, ch] + p_ref[1, ch] + v_ref[ch]) * dis
        for half in range(2):
            ab = a[:, half * 64:(half + 1) * 64]
            g = jnp.dot(ab, w1, preferred_element_type=jnp.float32) + b1
            g = jnp.maximum(g, 0.0)
            u = jnp.dot(g, w2, preferred_element_type=jnp.float32) * dis
            o_ref[2 * ch + half] = u


def _run_mid(p, v, dis, w1, b1, w2):
    grid = (NP // RT,)
    return pl.pallas_call(
        _mid_body,
        grid=grid,
        in_specs=[
            pl.BlockSpec((2, 4, RT, 128), lambda r: (0, 0, r, 0)),
            pl.BlockSpec((4, RT, 128), lambda r: (0, r, 0)),
            pl.BlockSpec((RT, 1), lambda r: (r, 0)),
            pl.BlockSpec((C, H), lambda r: (0, 0)),
            pl.BlockSpec((1, H), lambda r: (0, 0)),
            pl.BlockSpec((H, O), lambda r: (0, 0)),
        ],
        out_specs=pl.BlockSpec((B, RT, 128), lambda r: (0, r, 0)),
        out_shape=jax.ShapeDtypeStruct((B, NP, 128), jnp.float32),
    )(p, v, dis, w1, b1, w2)


def _head_body(p_ref, v2_ref, dis_ref, b2_ref, mw1_ref, mb1_ref, mw2_ref,
               mb2_ref, o_ref):
    dis = dis_ref[...]
    b2 = b2_ref[...]
    mw1 = mw1_ref[...]
    mb1 = mb1_ref[...]
    mw2 = mw2_ref[...]
    mb2 = mb2_ref[...]
    for b in range(B):
        x = (p_ref[0, b] + p_ref[1, b] + v2_ref[b]) * dis + b2
        t = jnp.dot(x, mw1, preferred_element_type=jnp.float32) + mb1
        t = jnp.maximum(t, 0.0)
        o_ref[b] = jnp.dot(t, mw2, preferred_element_type=jnp.float32) + mb2


def _run_head(p, v2, dis, b2, mw1, mb1, mw2, mb2):
    grid = (NP // RT,)
    return pl.pallas_call(
        _head_body,
        grid=grid,
        in_specs=[
            pl.BlockSpec((2, B, RT, 128), lambda r: (0, 0, r, 0)),
            pl.BlockSpec((B, RT, 128), lambda r: (0, r, 0)),
            pl.BlockSpec((RT, 1), lambda r: (r, 0)),
            pl.BlockSpec((1, O), lambda r: (0, 0)),
            pl.BlockSpec((O, M), lambda r: (0, 0)),
            pl.BlockSpec((1, M), lambda r: (0, 0)),
            pl.BlockSpec((M, NC), lambda r: (0, 0)),
            pl.BlockSpec((1, NC), lambda r: (0, 0)),
        ],
        out_specs=pl.BlockSpec((B, RT, NC), lambda r: (0, r, 0)),
        out_shape=jax.ShapeDtypeStruct((B, NP, NC), jnp.float32),
    )(p, v2, dis, b2, mw1, mb1, mw2, mb2)


def kernel(price_data_x, edge_index, conv_w, conv_b, gcn_w1, gcn_b1, gcn_w2,
           gcn_b2, mlp_w1, mlp_b1, mlp_w2, mlp_b2):
    ei3 = edge_index.reshape(2, NW, NBLK, EBLK)

    deg_parts = _deg_kernel(ei3)
    feat = _run_cnn(price_data_x, conv_w, conv_b)

    degT = jnp.transpose(deg_parts, (1, 0))
    featT = jnp.transpose(feat, (2, 0, 1)).reshape(NP, B * C)
    v, dis = _run_scale(degT, featT)

    agg1 = _agg4(v, ei3)
    v2 = _run_mid(agg1, v, dis, gcn_w1, gcn_b1.reshape(1, H), gcn_w2)
    agg2 = _agg8(v2, ei3)
    out = _run_head(agg2, v2, dis, gcn_b2.reshape(1, O),
                    mlp_w1, mlp_b1.reshape(1, M), mlp_w2,
                    mlp_b2.reshape(1, NC))
    return out[:, :N, :]


# speedup vs baseline: 1.1324x; 1.1324x over previous
import functools

import jax
import jax.numpy as jnp
from jax import lax
from jax.experimental import pallas as pl
from jax.experimental.pallas import tpu as pltpu
from jax.experimental.pallas import tpu_sc as plsc

B, S, N, E = 8, 256, 10000, 160000
C, H, O, M, NC = 64, 256, 128, 256, 2
NP = 10240
NW = 32
EPW = E // NW
EBLK = 100
NBLK = EPW // EBLK
ROWS_PW = NP // 16
NT = 512
RT = 256

_mesh = plsc.VectorSubcoreMesh(core_axis_name="c", subcore_axis_name="s")


def _zero_rows(zbuf_v, ncols):
    nregs = ncols // 16
    z = jnp.zeros((16,), jnp.float32)

    def body(i, _):
        for k in range(nregs):
            zbuf_v[i, pl.ds(k * 16, 16)] = z
        return 0

    lax.fori_loop(0, zbuf_v.shape[0], body, 0)


@functools.partial(
    pl.kernel, mesh=_mesh,
    out_type=jax.ShapeDtypeStruct((2, NP), jnp.float32),
    scratch_types=[
        pltpu.VMEM((NBLK, EBLK), jnp.int32),
        pltpu.VMEM((112,), jnp.float32),
        pltpu.VMEM((ROWS_PW,), jnp.float32),
        pltpu.VMEM_SHARED((NP,), jnp.float32),
    ],
)
def _deg_kernel(ei_hbm, out_hbm, idx_v, ones_v, zrow_v, acc_sh):
    c = lax.axis_index("c")
    s = lax.axis_index("s")
    w = c * 16 + s
    one = jnp.full((16,), 1.0, jnp.float32)
    z = jnp.zeros((16,), jnp.float32)
    for i in range(7):
        ones_v[pl.ds(i * 16, 16)] = one
    for i in range(ROWS_PW // 16):
        zrow_v[pl.ds(i * 16, 16)] = z
    pltpu.sync_copy(zrow_v, acc_sh.at[pl.ds(s * ROWS_PW, ROWS_PW)])
    plsc.subcore_barrier()
    pltpu.sync_copy(ei_hbm.at[1, w], idx_v)

    def blk(j, _):
        pltpu.sync_copy(ones_v.at[pl.ds(0, EBLK)], acc_sh.at[idx_v.at[j]],
                        add=True)
        return 0

    lax.fori_loop(0, NBLK, blk, 0)
    plsc.subcore_barrier()
    pltpu.sync_copy(acc_sh.at[pl.ds(s * ROWS_PW, ROWS_PW)],
                    out_hbm.at[c, pl.ds(s * ROWS_PW, ROWS_PW)])


def _make_agg(f_chunks):
    @functools.partial(
        pl.kernel, mesh=_mesh,
        out_type=jax.ShapeDtypeStruct((2, f_chunks, NP, 128), jnp.float32),
        scratch_types=[
            pltpu.VMEM((NBLK, EBLK), jnp.int32),
            pltpu.VMEM((NBLK, EBLK), jnp.int32),
            pltpu.VMEM((EBLK, 128), jnp.float32),
            pltpu.VMEM((EBLK, 128), jnp.float32),
            pltpu.VMEM((32, 128), jnp.float32),
            pltpu.VMEM_SHARED((NP, 128), jnp.float32),
            pltpu.SemaphoreType.DMA,
            pltpu.SemaphoreType.DMA,
        ],
    )
    def agg(v_hbm, ei_hbm, out_hbm, src_v, dst_v, buf0_v, buf1_v, zbuf_v,
            acc_sh, sem0, sem1):
        c = lax.axis_index("c")
        s = lax.axis_index("s")
        w = c * 16 + s
        _zero_rows(zbuf_v, 128)
        pltpu.sync_copy(ei_hbm.at[0, w], src_v)
        pltpu.sync_copy(ei_hbm.at[1, w], dst_v)
        for ch in range(f_chunks):
            vch = v_hbm.at[ch]
            for zz in range(ROWS_PW // 32):
                pltpu.sync_copy(
                    zbuf_v, acc_sh.at[pl.ds(s * ROWS_PW + zz * 32, 32)])
            plsc.subcore_barrier()

            bufs = (buf0_v, buf1_v)
            sems = (sem0, sem1)
            pltpu.async_copy(vch.at[src_v.at[0]], buf0_v, sem0)
            pltpu.async_copy(vch.at[src_v.at[1]], buf1_v, sem1)

            def blk(jj, _):
                for b2 in range(2):
                    j = jj * 2 + b2
                    buf, sem = bufs[b2], sems[b2]
                    pltpu.make_async_copy(vch.at[src_v.at[j]], buf, sem).wait()
                    pltpu.sync_copy(buf, acc_sh.at[dst_v.at[j]], add=True)
                    nxt = j + 2

                    @pl.when(nxt < NBLK)
                    def _():
                        pltpu.async_copy(vch.at[src_v.at[nxt]], buf, sem)
                return 0

            lax.fori_loop(0, NBLK // 2, blk, 0)
            plsc.subcore_barrier()
            pltpu.sync_copy(acc_sh.at[pl.ds(s * ROWS_PW, ROWS_PW)],
                            out_hbm.at[c, ch, pl.ds(s * ROWS_PW, ROWS_PW)])
    return agg


_agg4 = _make_agg(4)
_agg8 = _make_agg(8)


def _cnn_body(x_ref, w_ref, b_ref, o_ref):
    x2 = x_ref[0]
    zrow = jnp.zeros((1, NT), jnp.float32)
    xm1 = jnp.concatenate([zrow, x2[:-1, :]], axis=0)
    xp1 = jnp.concatenate([x2[1:, :], zrow], axis=0)
    inv_s = 1.0 / S
    for c in range(C):
        w0 = w_ref[c, 0, 0]
        w1 = w_ref[c, 0, 1]
        w2 = w_ref[c, 0, 2]
        h = xm1 * w0 + x2 * w1 + xp1 * w2 + b_ref[c]
        h = jnp.maximum(h, 0.0)
        o_ref[0, c, :] = jnp.sum(h, axis=0) * inv_s


def _run_cnn(x, conv_w, conv_b):
    grid = (B, NP // NT)
    return pl.pallas_call(
        _cnn_body,
        grid=grid,
        in_specs=[
            pl.BlockSpec((1, S, NT), lambda b, n: (b, 0, n)),
            pl.BlockSpec(memory_space=pltpu.SMEM),
            pl.BlockSpec(memory_space=pltpu.SMEM),
        ],
        out_specs=pl.BlockSpec((1, C, NT), lambda b, n: (b, 0, n)),
        out_shape=jax.ShapeDtypeStruct((B, C, NP), jnp.float32),
    )(x, conv_w, conv_b)


def _scale_body(degT_ref, feat_ref, v_ref, dis_ref):
    deg = degT_ref[:, 0:1] + degT_ref[:, 1:2] + 1.0
    dis = lax.rsqrt(deg)
    dis_ref[...] = dis
    for ch in range(4):
        b0 = 2 * ch
        t0 = jnp.transpose(feat_ref[b0], (1, 0)) * dis
        t1 = jnp.transpose(feat_ref[b0 + 1], (1, 0)) * dis
        v_ref[ch] = jnp.concatenate([t0, t1], axis=1)


def _run_scale(degT, feat):
    grid = (NP // RT,)
    return pl.pallas_call(
        _scale_body,
        grid=grid,
        in_specs=[
            pl.BlockSpec((RT, 2), lambda r: (r, 0)),
            pl.BlockSpec((B, C, RT), lambda r: (0, 0, r)),
        ],
        out_specs=[
            pl.BlockSpec((4, RT, 128), lambda r: (0, r, 0)),
            pl.BlockSpec((RT, 1), lambda r: (r, 0)),
        ],
        out_shape=[
            jax.ShapeDtypeStruct((4, NP, 128), jnp.float32),
            jax.ShapeDtypeStruct((NP, 1), jnp.float32),
        ],
    )(degT, feat)


def _mid_body(p_ref, v_ref, dis_ref, w1_ref, b1_ref, w2_ref, o_ref):
    dis = dis_ref[...]
    w1 = w1_ref[...]
    b1 = b1_ref[...]
    w2 = w2_ref[...]
    for ch in range(4):
        a = (p_ref[0, ch] + p_ref[1, ch] + v_ref[ch]) * dis
        for half in range(2):
            ab = a[:, half * 64:(half + 1) * 64]
            g = jnp.dot(ab, w1, preferred_element_type=jnp.float32) + b1
            g = jnp.maximum(g, 0.0)
            u = jnp.dot(g, w2, preferred_element_type=jnp.float32) * dis
            o_ref[2 * ch + half] = u


def _run_mid(p, v, dis, w1, b1, w2):
    grid = (NP // RT,)
    return pl.pallas_call(
        _mid_body,
        grid=grid,
        in_specs=[
            pl.BlockSpec((2, 4, RT, 128), lambda r: (0, 0, r, 0)),
            pl.BlockSpec((4, RT, 128), lambda r: (0, r, 0)),
            pl.BlockSpec((RT, 1), lambda r: (r, 0)),
            pl.BlockSpec((C, H), lambda r: (0, 0)),
            pl.BlockSpec((1, H), lambda r: (0, 0)),
            pl.BlockSpec((H, O), lambda r: (0, 0)),
        ],
        out_specs=pl.BlockSpec((B, RT, 128), lambda r: (0, r, 0)),
        out_shape=jax.ShapeDtypeStruct((B, NP, 128), jnp.float32),
    )(p, v, dis, w1, b1, w2)


def _head_body(p_ref, v2_ref, dis_ref, b2_ref, mw1_ref, mb1_ref, mw2_ref,
               mb2_ref, o_ref):
    dis = dis_ref[...]
    b2 = b2_ref[...]
    mw1 = mw1_ref[...]
    mb1 = mb1_ref[...]
    mw2 = mw2_ref[...]
    mb2 = mb2_ref[...]
    for b in range(B):
        x = (p_ref[0, b] + p_ref[1, b] + v2_ref[b]) * dis + b2
        t = jnp.dot(x, mw1, preferred_element_type=jnp.float32) + mb1
        t = jnp.maximum(t, 0.0)
        o_ref[b] = jnp.dot(t, mw2, preferred_element_type=jnp.float32) + mb2


def _run_head(p, v2, dis, b2, mw1, mb1, mw2, mb2):
    grid = (NP // RT,)
    return pl.pallas_call(
        _head_body,
        grid=grid,
        in_specs=[
            pl.BlockSpec((2, B, RT, 128), lambda r: (0, 0, r, 0)),
            pl.BlockSpec((B, RT, 128), lambda r: (0, r, 0)),
            pl.BlockSpec((RT, 1), lambda r: (r, 0)),
            pl.BlockSpec((1, O), lambda r: (0, 0)),
            pl.BlockSpec((O, M), lambda r: (0, 0)),
            pl.BlockSpec((1, M), lambda r: (0, 0)),
            pl.BlockSpec((M, NC), lambda r: (0, 0)),
            pl.BlockSpec((1, NC), lambda r: (0, 0)),
        ],
        out_specs=pl.BlockSpec((B, RT, NC), lambda r: (0, r, 0)),
        out_shape=jax.ShapeDtypeStruct((B, NP, NC), jnp.float32),
    )(p, v2, dis, b2, mw1, mb1, mw2, mb2)


def kernel(price_data_x, edge_index, conv_w, conv_b, gcn_w1, gcn_b1, gcn_w2,
           gcn_b2, mlp_w1, mlp_b1, mlp_w2, mlp_b2):
    ei3 = edge_index.reshape(2, NW, NBLK, EBLK)

    deg_parts = _deg_kernel(ei3)
    feat = _run_cnn(price_data_x, conv_w, conv_b)

    degT = jnp.transpose(deg_parts, (1, 0))
    v, dis = _run_scale(degT, feat)

    agg1 = _agg4(v, ei3)
    v2 = _run_mid(agg1, v, dis, gcn_w1, gcn_b1.reshape(1, H), gcn_w2)
    agg2 = _agg8(v2, ei3)
    out = _run_head(agg2, v2, dis, gcn_b2.reshape(1, O),
                    mlp_w1, mlp_b1.reshape(1, M), mlp_w2,
                    mlp_b2.reshape(1, NC))
    return out[:, :N, :]


# speedup vs baseline: 2.1725x; 1.9185x over previous
import functools

import jax
import jax.numpy as jnp
from jax import lax
from jax.experimental import pallas as pl
from jax.experimental.pallas import tpu as pltpu
from jax.experimental.pallas import tpu_sc as plsc

B, S, N, E = 8, 256, 10000, 160000
C, H, O, M, NC = 64, 256, 128, 256, 2
NP = 10240
NW = 32
EPW = E // NW
EBLK = 100
NBLK = EPW // EBLK
ROWS_PW = NP // 16
NT = 512
RT = 256

_mesh = plsc.VectorSubcoreMesh(core_axis_name="c", subcore_axis_name="s")


def _zero_rows(zbuf_v, ncols):
    nregs = ncols // 16
    z = jnp.zeros((16,), jnp.float32)

    def body(i, _):
        for k in range(nregs):
            zbuf_v[i, pl.ds(k * 16, 16)] = z
        return 0

    lax.fori_loop(0, zbuf_v.shape[0], body, 0)


@functools.partial(
    pl.kernel, mesh=_mesh,
    out_type=jax.ShapeDtypeStruct((2, NP), jnp.float32),
    scratch_types=[
        pltpu.VMEM((NBLK, EBLK), jnp.int32),
        pltpu.VMEM((112,), jnp.float32),
        pltpu.VMEM((ROWS_PW,), jnp.float32),
        pltpu.VMEM_SHARED((NP,), jnp.float32),
    ],
)
def _deg_kernel(ei_hbm, out_hbm, idx_v, ones_v, zrow_v, acc_sh):
    c = lax.axis_index("c")
    s = lax.axis_index("s")
    w = c * 16 + s
    one = jnp.full((16,), 1.0, jnp.float32)
    z = jnp.zeros((16,), jnp.float32)
    for i in range(7):
        ones_v[pl.ds(i * 16, 16)] = one
    for i in range(ROWS_PW // 16):
        zrow_v[pl.ds(i * 16, 16)] = z
    pltpu.sync_copy(zrow_v, acc_sh.at[pl.ds(s * ROWS_PW, ROWS_PW)])
    plsc.subcore_barrier()
    pltpu.sync_copy(ei_hbm.at[1, w], idx_v)

    def blk(j, _):
        pltpu.sync_copy(ones_v.at[pl.ds(0, EBLK)], acc_sh.at[idx_v.at[j]],
                        add=True)
        return 0

    lax.fori_loop(0, NBLK, blk, 0)
    plsc.subcore_barrier()
    pltpu.sync_copy(acc_sh.at[pl.ds(s * ROWS_PW, ROWS_PW)],
                    out_hbm.at[c, pl.ds(s * ROWS_PW, ROWS_PW)])


def _make_agg(f_chunks):
    @functools.partial(
        pl.kernel, mesh=_mesh,
        out_type=jax.ShapeDtypeStruct((2, f_chunks, NP, 128), jnp.float32),
        scratch_types=[
            pltpu.VMEM((NBLK, EBLK), jnp.int32),
            pltpu.VMEM((NBLK, EBLK), jnp.int32),
            pltpu.VMEM((EBLK, 128), jnp.float32),
            pltpu.VMEM((EBLK, 128), jnp.float32),
            pltpu.VMEM((32, 128), jnp.float32),
            pltpu.VMEM_SHARED((NP, 128), jnp.float32),
            pltpu.SemaphoreType.DMA,
            pltpu.SemaphoreType.DMA,
        ],
    )
    def agg(v_hbm, ei_hbm, out_hbm, src_v, dst_v, buf0_v, buf1_v, zbuf_v,
            acc_sh, sem0, sem1):
        c = lax.axis_index("c")
        s = lax.axis_index("s")
        w = c * 16 + s
        _zero_rows(zbuf_v, 128)
        pltpu.sync_copy(ei_hbm.at[0, w], src_v)
        pltpu.sync_copy(ei_hbm.at[1, w], dst_v)
        for ch in range(f_chunks):
            vch = v_hbm.at[ch]
            for zz in range(ROWS_PW // 32):
                pltpu.sync_copy(
                    zbuf_v, acc_sh.at[pl.ds(s * ROWS_PW + zz * 32, 32)])
            plsc.subcore_barrier()

            bufs = (buf0_v, buf1_v)
            sems = (sem0, sem1)
            pltpu.async_copy(vch.at[src_v.at[0]], buf0_v, sem0)
            pltpu.async_copy(vch.at[src_v.at[1]], buf1_v, sem1)

            def blk(jj, _):
                for b2 in range(2):
                    j = jj * 2 + b2
                    buf, sem = bufs[b2], sems[b2]
                    pltpu.make_async_copy(vch.at[src_v.at[j]], buf, sem).wait()
                    pltpu.sync_copy(buf, acc_sh.at[dst_v.at[j]], add=True)
                    nxt = j + 2

                    @pl.when(nxt < NBLK)
                    def _():
                        pltpu.async_copy(vch.at[src_v.at[nxt]], buf, sem)
                return 0

            lax.fori_loop(0, NBLK // 2, blk, 0)
            plsc.subcore_barrier()
            pltpu.sync_copy(acc_sh.at[pl.ds(s * ROWS_PW, ROWS_PW)],
                            out_hbm.at[c, ch, pl.ds(s * ROWS_PW, ROWS_PW)])
    return agg


_agg4 = _make_agg(4)
_agg8 = _make_agg(8)


TBLK = 128
KW = 144


def _band_weights(conv_w, conv_b):
    tl = jnp.arange(TBLK)
    u = jnp.arange(KW)
    k = u[None, :] - tl[:, None]
    valid = (k >= 0) & (k <= 2)
    kc = jnp.clip(k, 0, 2)
    w3 = conv_w[:, 0, :]
    taps = w3[:, kc]
    wb = jnp.where(valid[:, None, :], jnp.transpose(taps, (1, 0, 2)), 0.0)
    wb = wb + (u == 136).astype(jnp.float32)[None, None, :] * \
        conv_b[None, :, None]
    return wb.reshape(TBLK * C, KW).astype(jnp.bfloat16)


def _cnn_body(x_ref, wb_ref, o_ref):
    xb = x_ref[0].astype(jnp.bfloat16)
    z1 = jnp.zeros((1, NT), jnp.bfloat16)
    z6 = jnp.zeros((6, NT), jnp.bfloat16)
    z7 = jnp.zeros((7, NT), jnp.bfloat16)
    one = jnp.ones((1, NT), jnp.bfloat16)
    xw0 = jnp.concatenate([z1, xb[0:129], z6, one, z7], axis=0)
    xw1 = jnp.concatenate([xb[127:256], z7, one, z7], axis=0)
    acc = jnp.zeros((C, NT), jnp.float32)
    for xw in (xw0, xw1):
        for chunk in range(16):
            wrows = wb_ref[pl.ds(chunk * 512, 512), :]
            h = jnp.dot(wrows, xw, preferred_element_type=jnp.float32)
            r = jnp.maximum(h, 0.0).reshape(8, C, NT)
            acc = acc + jnp.sum(r, axis=0)
    o_ref[0] = acc * (1.0 / S)


def _run_cnn(x, wband):
    grid = (B, NP // NT)
    return pl.pallas_call(
        _cnn_body,
        grid=grid,
        in_specs=[
            pl.BlockSpec((1, S, NT), lambda b, n: (b, 0, n)),
            pl.BlockSpec((TBLK * C, KW), lambda b, n: (0, 0)),
        ],
        out_specs=pl.BlockSpec((1, C, NT), lambda b, n: (b, 0, n)),
        out_shape=jax.ShapeDtypeStruct((B, C, NP), jnp.float32),
    )(x, wband)


def _scale_body(degT_ref, feat_ref, v_ref, dis_ref):
    deg = degT_ref[:, 0:1] + degT_ref[:, 1:2] + 1.0
    dis = lax.rsqrt(deg)
    dis_ref[...] = dis
    for ch in range(4):
        b0 = 2 * ch
        t0 = jnp.transpose(feat_ref[b0], (1, 0)) * dis
        t1 = jnp.transpose(feat_ref[b0 + 1], (1, 0)) * dis
        v_ref[ch] = jnp.concatenate([t0, t1], axis=1)


def _run_scale(degT, feat):
    grid = (NP // RT,)
    return pl.pallas_call(
        _scale_body,
        grid=grid,
        in_specs=[
            pl.BlockSpec((RT, 2), lambda r: (r, 0)),
            pl.BlockSpec((B, C, RT), lambda r: (0, 0, r)),
        ],
        out_specs=[
            pl.BlockSpec((4, RT, 128), lambda r: (0, r, 0)),
            pl.BlockSpec((RT, 1), lambda r: (r, 0)),
        ],
        out_shape=[
            jax.ShapeDtypeStruct((4, NP, 128), jnp.float32),
            jax.ShapeDtypeStruct((NP, 1), jnp.float32),
        ],
    )(degT, feat)


def _mid_body(p_ref, v_ref, dis_ref, w1_ref, b1_ref, w2_ref, o_ref):
    dis = dis_ref[...]
    w1 = w1_ref[...]
    b1 = b1_ref[...]
    w2 = w2_ref[...]
    for ch in range(4):
        a = (p_ref[0, ch] + p_ref[1, ch] + v_ref[ch]) * dis
        for half in range(2):
            ab = a[:, half * 64:(half + 1) * 64]
            g = jnp.dot(ab, w1, preferred_element_type=jnp.float32) + b1
            g = jnp.maximum(g, 0.0)
            u = jnp.dot(g, w2, preferred_element_type=jnp.float32) * dis
            o_ref[2 * ch + half] = u


def _run_mid(p, v, dis, w1, b1, w2):
    grid = (NP // RT,)
    return pl.pallas_call(
        _mid_body,
        grid=grid,
        in_specs=[
            pl.BlockSpec((2, 4, RT, 128), lambda r: (0, 0, r, 0)),
            pl.BlockSpec((4, RT, 128), lambda r: (0, r, 0)),
            pl.BlockSpec((RT, 1), lambda r: (r, 0)),
            pl.BlockSpec((C, H), lambda r: (0, 0)),
            pl.BlockSpec((1, H), lambda r: (0, 0)),
            pl.BlockSpec((H, O), lambda r: (0, 0)),
        ],
        out_specs=pl.BlockSpec((B, RT, 128), lambda r: (0, r, 0)),
        out_shape=jax.ShapeDtypeStruct((B, NP, 128), jnp.float32),
    )(p, v, dis, w1, b1, w2)


def _head_body(p_ref, v2_ref, dis_ref, b2_ref, mw1_ref, mb1_ref, mw2_ref,
               mb2_ref, o_ref):
    dis = dis_ref[...]
    b2 = b2_ref[...]
    mw1 = mw1_ref[...]
    mb1 = mb1_ref[...]
    mw2 = mw2_ref[...]
    mb2 = mb2_ref[...]
    for b in range(B):
        x = (p_ref[0, b] + p_ref[1, b] + v2_ref[b]) * dis + b2
        t = jnp.dot(x, mw1, preferred_element_type=jnp.float32) + mb1
        t = jnp.maximum(t, 0.0)
        o_ref[b] = jnp.dot(t, mw2, preferred_element_type=jnp.float32) + mb2


def _run_head(p, v2, dis, b2, mw1, mb1, mw2, mb2):
    grid = (NP // RT,)
    return pl.pallas_call(
        _head_body,
        grid=grid,
        in_specs=[
            pl.BlockSpec((2, B, RT, 128), lambda r: (0, 0, r, 0)),
            pl.BlockSpec((B, RT, 128), lambda r: (0, r, 0)),
            pl.BlockSpec((RT, 1), lambda r: (r, 0)),
            pl.BlockSpec((1, O), lambda r: (0, 0)),
            pl.BlockSpec((O, M), lambda r: (0, 0)),
            pl.BlockSpec((1, M), lambda r: (0, 0)),
            pl.BlockSpec((M, NC), lambda r: (0, 0)),
            pl.BlockSpec((1, NC), lambda r: (0, 0)),
        ],
        out_specs=pl.BlockSpec((B, RT, NC), lambda r: (0, r, 0)),
        out_shape=jax.ShapeDtypeStruct((B, NP, NC), jnp.float32),
    )(p, v2, dis, b2, mw1, mb1, mw2, mb2)


def kernel(price_data_x, edge_index, conv_w, conv_b, gcn_w1, gcn_b1, gcn_w2,
           gcn_b2, mlp_w1, mlp_b1, mlp_w2, mlp_b2):
    ei3 = edge_index.reshape(2, NW, NBLK, EBLK)

    deg_parts = _deg_kernel(ei3)
    wband = _band_weights(conv_w, conv_b)
    feat = _run_cnn(price_data_x, wband)

    degT = jnp.transpose(deg_parts, (1, 0))
    v, dis = _run_scale(degT, feat)

    agg1 = _agg4(v, ei3)
    v2 = _run_mid(agg1, v, dis, gcn_w1, gcn_b1.reshape(1, H), gcn_w2)
    agg2 = _agg8(v2, ei3)
    out = _run_head(agg2, v2, dis, gcn_b2.reshape(1, O),
                    mlp_w1, mlp_b1.reshape(1, M), mlp_w2,
                    mlp_b2.reshape(1, NC))
    return out[:, :N, :]
